# Initial kernel scaffold; baseline (speedup 1.0000x reference)
import jax
import jax.numpy as jnp
from jax.experimental import pallas as pl


def kernel(pred_r, pred_t, target, model_points, idx):
    raise NotImplementedError("write your pallas kernel here")


# speedup vs baseline: 2.6981x; 2.6981x over previous
import jax
import jax.numpy as jnp
from jax.experimental import pallas as pl
from jax.experimental.pallas import tpu as pltpu

_BS = 64
_N = 3000
_NPAD = 3072
_I_TILE = 512
_N_IT = _NPAD // _I_TILE
_SYM = (12, 15, 18, 19, 20)
_PADVAL = 1e15


def _loss_kernel(params_ref, mp_ref, tgt_ref, tgtT_ref, out_ref):
    it = pl.program_id(1)

    @pl.when(it == 0)
    def _init():
        out_ref[0] = jnp.zeros((1, 1), jnp.float32)

    mp = mp_ref[0]
    mx = mp[:, 0:1]
    my = mp[:, 1:2]
    mz = mp[:, 2:3]
    tfx = mx * params_ref[0, 0, 0] + my * params_ref[0, 0, 3] + mz * params_ref[0, 0, 6] + params_ref[0, 0, 9]
    tfy = mx * params_ref[0, 0, 1] + my * params_ref[0, 0, 4] + mz * params_ref[0, 0, 7] + params_ref[0, 0, 10]
    tfz = mx * params_ref[0, 0, 2] + my * params_ref[0, 0, 5] + mz * params_ref[0, 0, 8] + params_ref[0, 0, 11]

    row = jax.lax.broadcasted_iota(jnp.int32, (_I_TILE, 1), 0) + it * _I_TILE
    valid = (row < _N).astype(jnp.float32)

    m = params_ref[0, 0, 12]

    @pl.when(m > 0.5)
    def _sym():
        tg = tgtT_ref[0]
        dx = tfx - tg[0:1, :]
        dy = tfy - tg[1:2, :]
        dz = tfz - tg[2:3, :]
        d2 = dx * dx + dy * dy + dz * dz
        mind = jnp.min(d2, axis=1, keepdims=True)
        s = jnp.sum(jnp.sqrt(mind) * valid, axis=0, keepdims=True)
        out_ref[0] += s

    @pl.when(m <= 0.5)
    def _plain():
        tg = tgt_ref[0]
        dx = tfx - tg[:, 0:1]
        dy = tfy - tg[:, 1:2]
        dz = tfz - tg[:, 2:3]
        d2 = dx * dx + dy * dy + dz * dz
        s = jnp.sum(jnp.sqrt(d2) * valid, axis=0, keepdims=True)
        out_ref[0] += s


def kernel(pred_r, pred_t, target, model_points, idx):
    bs, num_p, _ = target.shape

    q = pred_r / jnp.linalg.norm(pred_r, axis=1, keepdims=True)
    w, x, y, z = q[:, 0], q[:, 1], q[:, 2], q[:, 3]
    r00 = 1.0 - 2.0 * (y * y + z * z)
    r01 = 2.0 * (x * y - w * z)
    r02 = 2.0 * (x * z + w * y)
    r10 = 2.0 * (x * y + w * z)
    r11 = 1.0 - 2.0 * (x * x + z * z)
    r12 = 2.0 * (y * z - w * x)
    r20 = 2.0 * (x * z - w * y)
    r21 = 2.0 * (y * z + w * x)
    r22 = 1.0 - 2.0 * (x * x + y * y)
    sym = jnp.asarray(_SYM, dtype=idx.dtype)
    mask = (idx[:, 0][:, None] == sym[None, :]).any(axis=1).astype(jnp.float32)
    zeros = jnp.zeros_like(w)
    params = jnp.stack(
        [r00, r01, r02, r10, r11, r12, r20, r21, r22,
         pred_t[:, 0], pred_t[:, 1], pred_t[:, 2], mask, zeros, zeros, zeros],
        axis=1).reshape(bs, 1, 16)

    pad_n = _NPAD - num_p
    mp_p = jnp.pad(model_points, ((0, 0), (0, pad_n), (0, 0)))
    tgt_p = jnp.pad(target, ((0, 0), (0, pad_n), (0, 0)))
    tgtT = jnp.pad(jnp.transpose(target, (0, 2, 1)),
                   ((0, 0), (0, 0), (0, pad_n)), constant_values=_PADVAL)

    out = pl.pallas_call(
        _loss_kernel,
        grid=(bs, _N_IT),
        in_specs=[
            pl.BlockSpec((1, 1, 16), lambda b, i: (b, 0, 0), memory_space=pltpu.SMEM),
            pl.BlockSpec((1, _I_TILE, 3), lambda b, i: (b, i, 0)),
            pl.BlockSpec((1, _I_TILE, 3), lambda b, i: (b, i, 0)),
            pl.BlockSpec((1, 3, _NPAD), lambda b, i: (b, 0, 0)),
        ],
        out_specs=pl.BlockSpec((1, 1, 1), lambda b, i: (b, 0, 0)),
        out_shape=jax.ShapeDtypeStruct((bs, 1, 1), jnp.float32),
    )(params, mp_p, tgt_p, tgtT)

    return out[:, 0, 0] / jnp.float32(num_p)


# speedup vs baseline: 4.9781x; 1.8451x over previous
import jax
import jax.numpy as jnp
from jax.experimental import pallas as pl
from jax.experimental.pallas import tpu as pltpu

_BS = 64
_N = 3000
_NPAD = 3072
_J_TILE = 512
_N_JT = _NPAD // _J_TILE
_SYM = (12, 15, 18, 19, 20)
_PADVAL = 1e15


def _loss_kernel(params_ref, mpT_ref, tgtT_ref, tgt_ref, out_ref):
    mpx = mpT_ref[0, 0:1, :]
    mpy = mpT_ref[0, 1:2, :]
    mpz = mpT_ref[0, 2:3, :]

    def p(k):
        return params_ref[0, 0, k]

    tfx = mpx * p(0) + mpy * p(3) + mpz * p(6) + p(9)
    tfy = mpx * p(1) + mpy * p(4) + mpz * p(7) + p(10)
    tfz = mpx * p(2) + mpy * p(5) + mpz * p(8) + p(11)

    lane = jax.lax.broadcasted_iota(jnp.int32, (1, _NPAD), 1)
    lvalid = (lane < _N).astype(jnp.float32)

    m = p(12)

    @pl.when(m > 0.5)
    def _sym():
        minacc = jnp.full((1, _NPAD), jnp.inf, dtype=jnp.float32)
        for jt in range(_N_JT):
            tg = tgt_ref[0, jt * _J_TILE:(jt + 1) * _J_TILE, :]
            dx = tg[:, 0:1] - tfx
            dy = tg[:, 1:2] - tfy
            dz = tg[:, 2:3] - tfz
            d2 = dx * dx + dy * dy + dz * dz
            minacc = jnp.minimum(minacc, jnp.min(d2, axis=0, keepdims=True))
        s = jnp.sum(jnp.sqrt(minacc) * lvalid, axis=1, keepdims=True)
        out_ref[0] = s

    @pl.when(m <= 0.5)
    def _plain():
        dx = tfx - tgtT_ref[0, 0:1, :]
        dy = tfy - tgtT_ref[0, 1:2, :]
        dz = tfz - tgtT_ref[0, 2:3, :]
        d2 = dx * dx + dy * dy + dz * dz
        s = jnp.sum(jnp.sqrt(d2) * lvalid, axis=1, keepdims=True)
        out_ref[0] = s


def kernel(pred_r, pred_t, target, model_points, idx):
    bs, num_p, _ = target.shape

    q = pred_r / jnp.linalg.norm(pred_r, axis=1, keepdims=True)
    w, x, y, z = q[:, 0], q[:, 1], q[:, 2], q[:, 3]
    r00 = 1.0 - 2.0 * (y * y + z * z)
    r01 = 2.0 * (x * y - w * z)
    r02 = 2.0 * (x * z + w * y)
    r10 = 2.0 * (x * y + w * z)
    r11 = 1.0 - 2.0 * (x * x + z * z)
    r12 = 2.0 * (y * z - w * x)
    r20 = 2.0 * (x * z - w * y)
    r21 = 2.0 * (y * z + w * x)
    r22 = 1.0 - 2.0 * (x * x + y * y)
    sym = jnp.asarray(_SYM, dtype=idx.dtype)
    mask = (idx[:, 0][:, None] == sym[None, :]).any(axis=1).astype(jnp.float32)
    zeros = jnp.zeros_like(w)
    params = jnp.stack(
        [r00, r01, r02, r10, r11, r12, r20, r21, r22,
         pred_t[:, 0], pred_t[:, 1], pred_t[:, 2], mask, zeros, zeros, zeros],
        axis=1).reshape(bs, 1, 16)

    pad_n = _NPAD - num_p
    mpT = jnp.pad(jnp.transpose(model_points, (0, 2, 1)),
                  ((0, 0), (0, 0), (0, pad_n)))
    tgtT = jnp.pad(jnp.transpose(target, (0, 2, 1)),
                   ((0, 0), (0, 0), (0, pad_n)), constant_values=_PADVAL)
    tgt_p = jnp.pad(target, ((0, 0), (0, pad_n), (0, 0)),
                    constant_values=_PADVAL)

    out = pl.pallas_call(
        _loss_kernel,
        grid=(bs,),
        in_specs=[
            pl.BlockSpec((1, 1, 16), lambda b: (b, 0, 0), memory_space=pltpu.SMEM),
            pl.BlockSpec((1, 3, _NPAD), lambda b: (b, 0, 0)),
            pl.BlockSpec((1, 3, _NPAD), lambda b: (b, 0, 0)),
            pl.BlockSpec((1, _NPAD, 3), lambda b: (b, 0, 0)),
        ],
        out_specs=pl.BlockSpec((1, 1, 1), lambda b: (b, 0, 0)),
        out_shape=jax.ShapeDtypeStruct((bs, 1, 1), jnp.float32),
    )(params, mpT, tgtT, tgt_p)

    return out[:, 0, 0] / jnp.float32(num_p)
